# Initial kernel scaffold; baseline (speedup 1.0000x reference)
import jax
import jax.numpy as jnp
from jax.experimental import pallas as pl


def kernel(h_t, current_position, dones, node_features, node_positions, adjacency_matrix, ptr, num_nodes, last_visited_node_idx):
    raise NotImplementedError("write your pallas kernel here")


# speedup vs baseline: 9.3015x; 9.3015x over previous
import jax
import jax.numpy as jnp
from jax.experimental import pallas as pl
from jax.experimental.pallas import tpu as pltpu

MEM = 4096
FEAT = 512
BQ = 16
TAU = 0.85
DMIN = 1.5
EPS = 1e-8
ROW_TILE = 256


def _simnorm_kernel(nf_ref, htT_ref, s_ref, fn2_ref):
    nf = nf_ref[...]
    s_ref[...] = jnp.dot(nf, htT_ref[...], preferred_element_type=jnp.float32)
    fn2_ref[...] = jnp.sum(nf * nf, axis=1, keepdims=True)


def _decide_kernel(S_ref, fn2_ref, ht_ref, cpos_ref, npos_ref, dones_ref,
                   lv_ref, scal_ref, cur_ref, pr_ref, pc_ref, pj_ref, ap_ref):
    S = S_ref[...]
    fn2 = fn2_ref[...]
    ht = ht_ref[...]
    G = jnp.dot(ht, ht.T, preferred_element_type=jnp.float32)
    cpos = cpos_ref[...]
    npos = npos_ref[...]
    dones = dones_ref[...]
    lv0 = lv_ref[...]
    ptr = scal_ref[0, 0]
    nn = scal_ref[0, 1]
    iota = jax.lax.broadcasted_iota(jnp.int32, (MEM, 1), 0)
    iota16 = jax.lax.broadcasted_iota(jnp.int32, (BQ, 1), 0)
    iota32 = jax.lax.broadcasted_iota(jnp.int32, (2 * BQ, 1), 0)
    fn = jnp.maximum(jnp.sqrt(fn2), EPS)
    neg_inf = jnp.float32(-jnp.inf)

    ev_row = []
    ev_a = []
    ev_b = []
    ev_pos = []
    ev_srow = []
    ev_n2 = []
    pair_r, pair_c, pair_j, pair_ap = [], [], [], []
    cur_list = []

    for i in range(BQ):
        done = dones[i, 0] != 0
        hn = jnp.maximum(jnp.sqrt(G[i, i]), EPS)
        col = S[:, i:i + 1] / (fn * hn)
        for j in range(i):
            dv = ev_a[j] * ev_srow[j][0, i] + jnp.sum(ev_b[j] * G[:, i][None, :])
            psim = dv / (jnp.maximum(jnp.sqrt(ev_n2[j]), EPS) * hn)
            col = jnp.where(iota == ev_row[j], psim, col)
        col = jnp.where(iota < nn, col, neg_inf)
        msim = jnp.max(col)
        msi = jnp.min(jnp.where(col == msim, iota, MEM)).astype(jnp.int32)
        pos_m = jnp.sum(jnp.where(iota == msi, npos, 0.0), axis=0, keepdims=True)
        for j in range(i):
            pos_m = jnp.where(ev_row[j] == msi, ev_pos[j], pos_m)
        dpos = cpos[i:i + 1, :] - pos_m
        dist = jnp.sqrt(jnp.sum(dpos * dpos))
        should_add = (msim < TAU) | (dist > DMIN)
        empty = (~done) & (nn < 1)
        active = (~done) & (nn >= 1)
        caseA = active & should_add & (nn < MEM)
        caseB = active & should_add & (nn >= MEM)
        caseC = active & (~should_add)
        write_idx = jnp.where(
            empty, 0,
            jnp.where(caseA, nn, jnp.where(caseB, ptr, msi))).astype(jnp.int32)
        a_cur = jnp.float32(1.0)
        b_cur = jnp.zeros((1, BQ), jnp.float32)
        for j in range(i):
            m = ev_row[j] == write_idx
            a_cur = jnp.where(m, ev_a[j], a_cur)
            b_cur = jnp.where(m, ev_b[j], b_cur)
        ei = (jax.lax.broadcasted_iota(jnp.int32, (1, BQ), 1) == i).astype(jnp.float32)
        a_new = jnp.where(caseC, 0.5 * a_cur, 0.0)
        b_new = jnp.where(caseC, 0.5 * b_cur, 0.0) + jnp.where(caseC, 0.5, 1.0) * ei
        wpos = jnp.where(caseC, 0.5 * pos_m + 0.5 * cpos[i:i + 1, :], cpos[i:i + 1, :])
        srow_w = jnp.sum(jnp.where(iota == write_idx, S, 0.0), axis=0, keepdims=True)
        fn2_w = jnp.sum(jnp.where(iota == write_idx, fn2, 0.0))
        bG = jnp.dot(b_new, G, preferred_element_type=jnp.float32)
        n2_new = (a_new * a_new * fn2_w + 2.0 * a_new * jnp.sum(b_new * srow_w)
                  + jnp.sum(bG * b_new))
        ev_row.append(jnp.where(done, -1, write_idx))
        ev_a.append(a_new)
        ev_b.append(b_new)
        ev_pos.append(wpos)
        ev_srow.append(srow_w)
        ev_n2.append(n2_new)
        nn = nn + jnp.where(empty | caseA, 1, 0)
        ptr = jnp.where(caseB, (ptr + 1) & (MEM - 1), ptr)
        cur = write_idx
        last_idx = jnp.where(empty, 0, lv0[i, 0])
        edge = (~done) & (last_idx != -1) & (last_idx != cur)
        for (r, c) in ((last_idx, cur), (cur, last_idx)):
            dup = jnp.bool_(False)
            for m in range(len(pair_r)):
                dup = dup | (pair_ap[m] & (pair_r[m] == r) & (pair_c[m] == c))
            pair_r.append(r)
            pair_c.append(c)
            pair_j.append(jnp.int32(i))
            pair_ap.append(edge & (~dup))
        cur_list.append(cur)

    cur_arr = jnp.zeros((BQ, 1), jnp.int32)
    for i in range(BQ):
        cur_arr = jnp.where(iota16 == i, cur_list[i], cur_arr)
    cur_ref[...] = cur_arr
    pr_a = jnp.zeros((2 * BQ, 1), jnp.int32)
    pc_a = jnp.zeros((2 * BQ, 1), jnp.int32)
    pj_a = jnp.zeros((2 * BQ, 1), jnp.int32)
    ap_a = jnp.zeros((2 * BQ, 1), jnp.int32)
    for k in range(2 * BQ):
        mk = iota32 == k
        pr_a = jnp.where(mk, pair_r[k], pr_a)
        pc_a = jnp.where(mk, pair_c[k], pc_a)
        pj_a = jnp.where(mk, pair_j[k], pj_a)
        ap_a = jnp.where(mk, pair_ap[k].astype(jnp.int32), ap_a)
    pr_ref[...] = pr_a
    pc_ref[...] = pc_a
    pj_ref[...] = pj_a
    ap_ref[...] = ap_a


def _gather_kernel(cur_sref, dn_sref, pr_sref, pc_sref, pj_sref, ap_sref,
                   adj_ref, topo_ref):
    i = pl.program_id(0)
    row = adj_ref[0, :, :]
    s = jnp.sum(row)
    ci = cur_sref[i]
    iota = jax.lax.broadcasted_iota(jnp.int32, (1, MEM), 1)
    corr = jnp.float32(0.0)
    for k in range(2 * BQ):
        base = jnp.sum(jnp.where(iota == pc_sref[k], row, 0.0))
        add = (ap_sref[k] != 0) & (pr_sref[k] == ci) & (pj_sref[k] <= i)
        corr = corr + jnp.where(add, 1.0 - base, 0.0)
    done = dn_sref[i] != 0
    val = jnp.where(done, 0.0, s + corr)
    topo_ref[...] = jnp.reshape(val, (1, 1, 1))


def kernel(h_t, current_position, dones, node_features, node_positions,
           adjacency_matrix, ptr, num_nodes, last_visited_node_idx):
    htT = h_t.T
    S, fn2 = pl.pallas_call(
        _simnorm_kernel,
        grid=(MEM // ROW_TILE,),
        in_specs=[
            pl.BlockSpec((ROW_TILE, FEAT), lambda i: (i, 0)),
            pl.BlockSpec((FEAT, BQ), lambda i: (0, 0)),
        ],
        out_specs=[
            pl.BlockSpec((ROW_TILE, BQ), lambda i: (i, 0)),
            pl.BlockSpec((ROW_TILE, 1), lambda i: (i, 0)),
        ],
        out_shape=[
            jax.ShapeDtypeStruct((MEM, BQ), jnp.float32),
            jax.ShapeDtypeStruct((MEM, 1), jnp.float32),
        ],
    )(node_features, htT)

    dones_i = dones.astype(jnp.int32).reshape(BQ, 1)
    lv = last_visited_node_idx.astype(jnp.int32).reshape(BQ, 1)
    scal = jnp.stack([ptr.astype(jnp.int32),
                      num_nodes.astype(jnp.int32)]).reshape(1, 2)
    cur, pr, pc, pj, ap = pl.pallas_call(
        _decide_kernel,
        out_shape=[
            jax.ShapeDtypeStruct((BQ, 1), jnp.int32),
            jax.ShapeDtypeStruct((2 * BQ, 1), jnp.int32),
            jax.ShapeDtypeStruct((2 * BQ, 1), jnp.int32),
            jax.ShapeDtypeStruct((2 * BQ, 1), jnp.int32),
            jax.ShapeDtypeStruct((2 * BQ, 1), jnp.int32),
        ],
    )(S, fn2, h_t, current_position, node_positions, dones_i, lv, scal)

    adj3 = adjacency_matrix.reshape(MEM, 1, MEM)
    grid_spec = pltpu.PrefetchScalarGridSpec(
        num_scalar_prefetch=6,
        grid=(BQ,),
        in_specs=[
            pl.BlockSpec((1, 1, MEM), lambda i, cur_s, dn, prs, pcs, pjs, aps:
                         (cur_s[i], 0, 0)),
        ],
        out_specs=pl.BlockSpec((1, 1, 1), lambda i, cur_s, dn, prs, pcs, pjs,
                               aps: (i, 0, 0)),
    )
    topo3 = pl.pallas_call(
        _gather_kernel,
        grid_spec=grid_spec,
        out_shape=jax.ShapeDtypeStruct((BQ, 1, 1), jnp.float32),
    )(cur.reshape(BQ), dones_i.reshape(BQ), pr.reshape(2 * BQ),
      pc.reshape(2 * BQ), pj.reshape(2 * BQ), ap.reshape(2 * BQ), adj3)
    return topo3.reshape(BQ, 1)


# speedup vs baseline: 14.9008x; 1.6020x over previous
import jax
import jax.numpy as jnp
from jax.experimental import pallas as pl
from jax.experimental.pallas import tpu as pltpu

MEM = 4096
FEAT = 512
BQ = 16
NP = 2 * BQ
TAU = 0.85
DMIN = 1.5
EPS = 1e-8
ROW_TILE = 512


def _simnorm_kernel(nf_ref, ht_ref, st_ref, fn2_ref):
    nf = nf_ref[...]
    dn = (((1,), (1,)), ((), ()))
    st_ref[...] = jax.lax.dot_general(
        ht_ref[...], nf, dimension_numbers=dn,
        preferred_element_type=jnp.float32)
    fn2_ref[...] = jax.lax.dot_general(
        jnp.ones((1, FEAT), jnp.float32), nf * nf, dimension_numbers=dn,
        preferred_element_type=jnp.float32)


def _decide_kernel(st_ref, fn2_ref, ht_ref, htT_ref, cpos_ref, npos_ref,
                   dones_ref, lv_ref, scal_ref, cur_ref, pr_ref, pc_ref,
                   pj_ref, ap_ref):
    ST = st_ref[...]
    fn2 = fn2_ref[...]
    ht = ht_ref[...]
    G = jnp.dot(ht, htT_ref[...], preferred_element_type=jnp.float32)
    cposT = cpos_ref[...]
    nposT = npos_ref[...]
    dones = dones_ref[...]
    lv0 = lv_ref[...]
    ptr = scal_ref[0, 0]
    nn = scal_ref[0, 1]
    iota = jax.lax.broadcasted_iota(jnp.int32, (1, MEM), 1)
    iota16 = jax.lax.broadcasted_iota(jnp.int32, (BQ, 1), 0)
    iota32 = jax.lax.broadcasted_iota(jnp.int32, (NP, 1), 0)
    fn = jnp.maximum(jnp.sqrt(fn2), EPS)
    neg_inf = jnp.float32(-jnp.inf)

    ev_row = []
    ev_a = []
    ev_b = []
    ev_pos = []
    ev_srow = []
    ev_n2 = []
    pair_r, pair_c, pair_j, pair_ap = [], [], [], []
    cur_list = []

    for i in range(BQ):
        done = dones[i, 0] != 0
        hn = jnp.maximum(jnp.sqrt(G[i, i]), EPS)
        col = ST[i:i + 1, :] / (fn * hn)
        for j in range(i):
            dv = ev_a[j] * ev_srow[j][i, 0] + jnp.sum(ev_b[j] * G[:, i][None, :])
            psim = dv / (jnp.maximum(jnp.sqrt(ev_n2[j]), EPS) * hn)
            col = jnp.where(iota == ev_row[j], psim, col)
        col = jnp.where(iota < nn, col, neg_inf)
        msim = jnp.max(col)
        msi = jnp.min(jnp.where(col == msim, iota, MEM)).astype(jnp.int32)
        pos_m = jnp.sum(jnp.where(iota == msi, nposT, 0.0), axis=1,
                        keepdims=True)
        for j in range(i):
            pos_m = jnp.where(ev_row[j] == msi, ev_pos[j], pos_m)
        dpos = cposT[:, i:i + 1] - pos_m
        dist = jnp.sqrt(jnp.sum(dpos * dpos))
        should_add = (msim < TAU) | (dist > DMIN)
        empty = (~done) & (nn < 1)
        active = (~done) & (nn >= 1)
        caseA = active & should_add & (nn < MEM)
        caseB = active & should_add & (nn >= MEM)
        caseC = active & (~should_add)
        write_idx = jnp.where(
            empty, 0,
            jnp.where(caseA, nn, jnp.where(caseB, ptr, msi))).astype(jnp.int32)
        a_cur = jnp.float32(1.0)
        b_cur = jnp.zeros((1, BQ), jnp.float32)
        for j in range(i):
            m = ev_row[j] == write_idx
            a_cur = jnp.where(m, ev_a[j], a_cur)
            b_cur = jnp.where(m, ev_b[j], b_cur)
        ei = (jax.lax.broadcasted_iota(jnp.int32, (1, BQ), 1) == i
              ).astype(jnp.float32)
        a_new = jnp.where(caseC, 0.5 * a_cur, 0.0)
        b_new = jnp.where(caseC, 0.5 * b_cur, 0.0) + jnp.where(caseC, 0.5, 1.0) * ei
        wpos = jnp.where(caseC, 0.5 * pos_m + 0.5 * cposT[:, i:i + 1],
                         cposT[:, i:i + 1])
        srow_w = jnp.sum(jnp.where(iota == write_idx, ST, 0.0), axis=1,
                         keepdims=True)
        fn2_w = jnp.sum(jnp.where(iota == write_idx, fn2, 0.0))
        bG = jnp.dot(b_new, G, preferred_element_type=jnp.float32)
        n2_new = (a_new * a_new * fn2_w
                  + 2.0 * a_new * jnp.dot(b_new, srow_w,
                                          preferred_element_type=jnp.float32)[0, 0]
                  + jnp.sum(bG * b_new))
        ev_row.append(jnp.where(done, -1, write_idx))
        ev_a.append(a_new)
        ev_b.append(b_new)
        ev_pos.append(wpos)
        ev_srow.append(srow_w)
        ev_n2.append(n2_new)
        nn = nn + jnp.where(empty | caseA, 1, 0)
        ptr = jnp.where(caseB, (ptr + 1) & (MEM - 1), ptr)
        cur = write_idx
        last_idx = jnp.where(empty, 0, lv0[i, 0])
        edge = (~done) & (last_idx != -1) & (last_idx != cur)
        for (r, c) in ((last_idx, cur), (cur, last_idx)):
            dup = jnp.bool_(False)
            for m in range(len(pair_r)):
                dup = dup | (pair_ap[m] & (pair_r[m] == r) & (pair_c[m] == c))
            pair_r.append(r)
            pair_c.append(c)
            pair_j.append(jnp.int32(i))
            pair_ap.append(edge & (~dup))
        cur_list.append(cur)

    cur_arr = jnp.zeros((BQ, 1), jnp.int32)
    for i in range(BQ):
        cur_arr = jnp.where(iota16 == i, cur_list[i], cur_arr)
    cur_ref[...] = cur_arr
    pr_a = jnp.zeros((NP, 1), jnp.int32)
    pc_a = jnp.zeros((NP, 1), jnp.int32)
    pj_a = jnp.zeros((NP, 1), jnp.int32)
    ap_a = jnp.zeros((NP, 1), jnp.int32)
    for k in range(NP):
        mk = iota32 == k
        pr_a = jnp.where(mk, pair_r[k], pr_a)
        pc_a = jnp.where(mk, pair_c[k], pc_a)
        pj_a = jnp.where(mk, pair_j[k], pj_a)
        ap_a = jnp.where(mk, pair_ap[k].astype(jnp.int32), ap_a)
    pr_ref[...] = pr_a
    pc_ref[...] = pc_a
    pj_ref[...] = pj_a
    ap_ref[...] = ap_a


def _gather_kernel(cur_sref, curv_ref, pr_ref, pc_ref, pj_ref, ap_ref,
                   dn_ref, adj_ref, topo_ref, rows_ref):
    i = pl.program_id(0)
    rows_ref[pl.ds(i, 1), :] = adj_ref[0, :, :]

    @pl.when(i == BQ - 1)
    def _assemble():
        rows = rows_ref[...]
        sums = jnp.sum(rows, axis=1, keepdims=True)
        curT = jnp.reshape(curv_ref[...], (1, BQ))
        pr = pr_ref[...]
        pc = pc_ref[...]
        pj = pj_ref[...]
        ap = ap_ref[...]
        m = (pr == curT).astype(jnp.float32)
        count = jnp.sum(m, axis=1, keepdims=True)
        sel = jnp.dot(m, rows, preferred_element_type=jnp.float32)
        iota_c = jax.lax.broadcasted_iota(jnp.int32, (1, MEM), 1)
        onehot_c = (iota_c == pc).astype(jnp.float32)
        bval = (jnp.sum(sel * onehot_c, axis=1, keepdims=True)
                / jnp.maximum(count, 1.0))
        iota_q = jax.lax.broadcasted_iota(jnp.int32, (1, BQ), 1)
        mask = ((ap != 0) & (pr == curT) & (pj <= iota_q)).astype(jnp.float32)
        corr = jnp.sum(mask * (1.0 - bval), axis=0, keepdims=True)
        corr16 = jnp.reshape(corr, (BQ, 1))
        topo = jnp.where(dn_ref[...] != 0, 0.0, sums + corr16)
        topo_ref[...] = topo


def kernel(h_t, current_position, dones, node_features, node_positions,
           adjacency_matrix, ptr, num_nodes, last_visited_node_idx):
    htT = h_t.T
    ST, fn2 = pl.pallas_call(
        _simnorm_kernel,
        grid=(MEM // ROW_TILE,),
        in_specs=[
            pl.BlockSpec((ROW_TILE, FEAT), lambda i: (i, 0)),
            pl.BlockSpec((BQ, FEAT), lambda i: (0, 0)),
        ],
        out_specs=[
            pl.BlockSpec((BQ, ROW_TILE), lambda i: (0, i)),
            pl.BlockSpec((1, ROW_TILE), lambda i: (0, i)),
        ],
        out_shape=[
            jax.ShapeDtypeStruct((BQ, MEM), jnp.float32),
            jax.ShapeDtypeStruct((1, MEM), jnp.float32),
        ],
    )(node_features, h_t)

    dones_i = dones.astype(jnp.int32).reshape(BQ, 1)
    lv = last_visited_node_idx.astype(jnp.int32).reshape(BQ, 1)
    scal = jnp.stack([ptr.astype(jnp.int32),
                      num_nodes.astype(jnp.int32)]).reshape(1, 2)
    cur, pr, pc, pj, ap = pl.pallas_call(
        _decide_kernel,
        out_shape=[
            jax.ShapeDtypeStruct((BQ, 1), jnp.int32),
            jax.ShapeDtypeStruct((NP, 1), jnp.int32),
            jax.ShapeDtypeStruct((NP, 1), jnp.int32),
            jax.ShapeDtypeStruct((NP, 1), jnp.int32),
            jax.ShapeDtypeStruct((NP, 1), jnp.int32),
        ],
    )(ST, fn2, h_t, htT, current_position.T, node_positions.T, dones_i, lv,
      scal)

    adj3 = adjacency_matrix.reshape(MEM, 1, MEM)
    full = lambda i, cur_s: (0, 0)
    grid_spec = pltpu.PrefetchScalarGridSpec(
        num_scalar_prefetch=1,
        grid=(BQ,),
        in_specs=[
            pl.BlockSpec((BQ, 1), full),
            pl.BlockSpec((NP, 1), full),
            pl.BlockSpec((NP, 1), full),
            pl.BlockSpec((NP, 1), full),
            pl.BlockSpec((NP, 1), full),
            pl.BlockSpec((BQ, 1), full),
            pl.BlockSpec((1, 1, MEM), lambda i, cur_s: (cur_s[i], 0, 0)),
        ],
        out_specs=pl.BlockSpec((BQ, 1), full),
        scratch_shapes=[pltpu.VMEM((BQ, MEM), jnp.float32)],
    )
    topo = pl.pallas_call(
        _gather_kernel,
        grid_spec=grid_spec,
        out_shape=jax.ShapeDtypeStruct((BQ, 1), jnp.float32),
    )(cur.reshape(BQ), cur, pr, pc, pj, ap, dones_i, adj3)
    return topo


# speedup vs baseline: 42.4434x; 2.8484x over previous
import jax
import jax.numpy as jnp
from jax.experimental import pallas as pl
from jax.experimental.pallas import tpu as pltpu

MEM = 4096
FEAT = 512
BQ = 16
NP = 2 * BQ
TAU = 0.85
DMIN = 1.5
EPS = 1e-8
ROW_TILE = 512


def _simnorm_kernel(nf_ref, ht_ref, st_ref, fn2_ref):
    nf = nf_ref[...]
    dn = (((1,), (1,)), ((), ()))
    st_ref[...] = jax.lax.dot_general(
        ht_ref[...], nf, dimension_numbers=dn,
        preferred_element_type=jnp.float32)
    fn2_ref[...] = jax.lax.dot_general(
        jnp.ones((1, FEAT), jnp.float32), nf * nf, dimension_numbers=dn,
        preferred_element_type=jnp.float32)


def _decide_kernel(st_ref, fn2_ref, ht_ref, htT_ref, cpos_ref, npos_ref,
                   dones_ref, lv_ref, scal_ref, cur_ref, pr_ref, pc_ref,
                   pj_ref, ap_ref):
    ST = st_ref[...]
    fn2 = fn2_ref[...]
    ht = ht_ref[...]
    G = jnp.dot(ht, htT_ref[...], preferred_element_type=jnp.float32)
    cposT = cpos_ref[...]
    nposT = npos_ref[...]
    dones = dones_ref[...]
    lv0 = lv_ref[...]
    ptr = scal_ref[0, 0]
    nn = scal_ref[0, 1]
    iota = jax.lax.broadcasted_iota(jnp.int32, (1, MEM), 1)
    iota16 = jax.lax.broadcasted_iota(jnp.int32, (BQ, 1), 0)
    iota32 = jax.lax.broadcasted_iota(jnp.int32, (NP, 1), 0)
    fn = jnp.maximum(jnp.sqrt(fn2), EPS)
    neg_inf = jnp.float32(-jnp.inf)

    ev_row = []
    ev_a = []
    ev_b = []
    ev_pos = []
    ev_srow = []
    ev_n2 = []
    pair_r, pair_c, pair_j, pair_ap = [], [], [], []
    cur_list = []

    for i in range(BQ):
        done = dones[i, 0] != 0
        hn = jnp.maximum(jnp.sqrt(G[i, i]), EPS)
        col = ST[i:i + 1, :] / (fn * hn)
        for j in range(i):
            dv = ev_a[j] * ev_srow[j][i, 0] + jnp.sum(ev_b[j] * G[:, i][None, :])
            psim = dv / (jnp.maximum(jnp.sqrt(ev_n2[j]), EPS) * hn)
            col = jnp.where(iota == ev_row[j], psim, col)
        col = jnp.where(iota < nn, col, neg_inf)
        msim = jnp.max(col)
        msi = jnp.min(jnp.where(col == msim, iota, MEM)).astype(jnp.int32)
        pos_m = jnp.sum(jnp.where(iota == msi, nposT, 0.0), axis=1,
                        keepdims=True)
        for j in range(i):
            pos_m = jnp.where(ev_row[j] == msi, ev_pos[j], pos_m)
        dpos = cposT[:, i:i + 1] - pos_m
        dist = jnp.sqrt(jnp.sum(dpos * dpos))
        should_add = (msim < TAU) | (dist > DMIN)
        empty = (~done) & (nn < 1)
        active = (~done) & (nn >= 1)
        caseA = active & should_add & (nn < MEM)
        caseB = active & should_add & (nn >= MEM)
        caseC = active & (~should_add)
        write_idx = jnp.where(
            empty, 0,
            jnp.where(caseA, nn, jnp.where(caseB, ptr, msi))).astype(jnp.int32)
        a_cur = jnp.float32(1.0)
        b_cur = jnp.zeros((1, BQ), jnp.float32)
        for j in range(i):
            m = ev_row[j] == write_idx
            a_cur = jnp.where(m, ev_a[j], a_cur)
            b_cur = jnp.where(m, ev_b[j], b_cur)
        ei = (jax.lax.broadcasted_iota(jnp.int32, (1, BQ), 1) == i
              ).astype(jnp.float32)
        a_new = jnp.where(caseC, 0.5 * a_cur, 0.0)
        b_new = jnp.where(caseC, 0.5 * b_cur, 0.0) + jnp.where(caseC, 0.5, 1.0) * ei
        wpos = jnp.where(caseC, 0.5 * pos_m + 0.5 * cposT[:, i:i + 1],
                         cposT[:, i:i + 1])
        srow_w = jnp.sum(jnp.where(iota == write_idx, ST, 0.0), axis=1,
                         keepdims=True)
        fn2_w = jnp.sum(jnp.where(iota == write_idx, fn2, 0.0))
        bG = jnp.dot(b_new, G, preferred_element_type=jnp.float32)
        n2_new = (a_new * a_new * fn2_w
                  + 2.0 * a_new * jnp.dot(b_new, srow_w,
                                          preferred_element_type=jnp.float32)[0, 0]
                  + jnp.sum(bG * b_new))
        ev_row.append(jnp.where(done, -1, write_idx))
        ev_a.append(a_new)
        ev_b.append(b_new)
        ev_pos.append(wpos)
        ev_srow.append(srow_w)
        ev_n2.append(n2_new)
        nn = nn + jnp.where(empty | caseA, 1, 0)
        ptr = jnp.where(caseB, (ptr + 1) & (MEM - 1), ptr)
        cur = write_idx
        last_idx = jnp.where(empty, 0, lv0[i, 0])
        edge = (~done) & (last_idx != -1) & (last_idx != cur)
        for (r, c) in ((last_idx, cur), (cur, last_idx)):
            dup = jnp.bool_(False)
            for m in range(len(pair_r)):
                dup = dup | (pair_ap[m] & (pair_r[m] == r) & (pair_c[m] == c))
            pair_r.append(r)
            pair_c.append(c)
            pair_j.append(jnp.int32(i))
            pair_ap.append(edge & (~dup))
        cur_list.append(cur)

    cur_arr = jnp.zeros((BQ, 1), jnp.int32)
    for i in range(BQ):
        cur_arr = jnp.where(iota16 == i, cur_list[i], cur_arr)
    cur_ref[...] = cur_arr
    pr_a = jnp.zeros((NP, 1), jnp.int32)
    pc_a = jnp.zeros((NP, 1), jnp.int32)
    pj_a = jnp.zeros((NP, 1), jnp.int32)
    ap_a = jnp.zeros((NP, 1), jnp.int32)
    for k in range(NP):
        mk = iota32 == k
        pr_a = jnp.where(mk, pair_r[k], pr_a)
        pc_a = jnp.where(mk, pair_c[k], pc_a)
        pj_a = jnp.where(mk, pair_j[k], pj_a)
        ap_a = jnp.where(mk, pair_ap[k].astype(jnp.int32), ap_a)
    pr_ref[...] = pr_a
    pc_ref[...] = pc_a
    pj_ref[...] = pj_a
    ap_ref[...] = ap_a


def _gather_kernel(cur_sref, curv_ref, pr_ref, pc_ref, pj_ref, ap_ref,
                   dn_ref, adj_ref, topo_ref, rows_ref):
    i = pl.program_id(0)
    sub = cur_sref[i] & 7
    iota8 = jax.lax.broadcasted_iota(jnp.int32, (8, 1), 0)
    rows_ref[pl.ds(i, 1), :] = jnp.sum(
        jnp.where(iota8 == sub, adj_ref[...], 0.0), axis=0, keepdims=True)

    @pl.when(i == BQ - 1)
    def _assemble():
        rows = rows_ref[...]
        sums = jnp.sum(rows, axis=1, keepdims=True)
        curT = jnp.reshape(curv_ref[...], (1, BQ))
        pr = pr_ref[...]
        pc = pc_ref[...]
        pj = pj_ref[...]
        ap = ap_ref[...]
        m = (pr == curT).astype(jnp.float32)
        count = jnp.sum(m, axis=1, keepdims=True)
        sel = jnp.dot(m, rows, preferred_element_type=jnp.float32)
        iota_c = jax.lax.broadcasted_iota(jnp.int32, (1, MEM), 1)
        onehot_c = (iota_c == pc).astype(jnp.float32)
        bval = (jnp.sum(sel * onehot_c, axis=1, keepdims=True)
                / jnp.maximum(count, 1.0))
        iota_q = jax.lax.broadcasted_iota(jnp.int32, (1, BQ), 1)
        mask = ((ap != 0) & (pr == curT) & (pj <= iota_q)).astype(jnp.float32)
        corr = jnp.sum(mask * (1.0 - bval), axis=0, keepdims=True)
        corr16 = jnp.reshape(corr, (BQ, 1))
        topo = jnp.where(dn_ref[...] != 0, 0.0, sums + corr16)
        topo_ref[...] = topo


def kernel(h_t, current_position, dones, node_features, node_positions,
           adjacency_matrix, ptr, num_nodes, last_visited_node_idx):
    htT = h_t.T
    ST, fn2 = pl.pallas_call(
        _simnorm_kernel,
        grid=(MEM // ROW_TILE,),
        in_specs=[
            pl.BlockSpec((ROW_TILE, FEAT), lambda i: (i, 0)),
            pl.BlockSpec((BQ, FEAT), lambda i: (0, 0)),
        ],
        out_specs=[
            pl.BlockSpec((BQ, ROW_TILE), lambda i: (0, i)),
            pl.BlockSpec((1, ROW_TILE), lambda i: (0, i)),
        ],
        out_shape=[
            jax.ShapeDtypeStruct((BQ, MEM), jnp.float32),
            jax.ShapeDtypeStruct((1, MEM), jnp.float32),
        ],
    )(node_features, h_t)

    dones_i = dones.astype(jnp.int32).reshape(BQ, 1)
    lv = last_visited_node_idx.astype(jnp.int32).reshape(BQ, 1)
    scal = jnp.stack([ptr.astype(jnp.int32),
                      num_nodes.astype(jnp.int32)]).reshape(1, 2)
    cur, pr, pc, pj, ap = pl.pallas_call(
        _decide_kernel,
        out_shape=[
            jax.ShapeDtypeStruct((BQ, 1), jnp.int32),
            jax.ShapeDtypeStruct((NP, 1), jnp.int32),
            jax.ShapeDtypeStruct((NP, 1), jnp.int32),
            jax.ShapeDtypeStruct((NP, 1), jnp.int32),
            jax.ShapeDtypeStruct((NP, 1), jnp.int32),
        ],
    )(ST, fn2, h_t, htT, current_position.T, node_positions.T, dones_i, lv,
      scal)

    full = lambda i, cur_s: (0, 0)
    grid_spec = pltpu.PrefetchScalarGridSpec(
        num_scalar_prefetch=1,
        grid=(BQ,),
        in_specs=[
            pl.BlockSpec((BQ, 1), full),
            pl.BlockSpec((NP, 1), full),
            pl.BlockSpec((NP, 1), full),
            pl.BlockSpec((NP, 1), full),
            pl.BlockSpec((NP, 1), full),
            pl.BlockSpec((BQ, 1), full),
            pl.BlockSpec((8, MEM), lambda i, cur_s: (cur_s[i] // 8, 0)),
        ],
        out_specs=pl.BlockSpec((BQ, 1), full),
        scratch_shapes=[pltpu.VMEM((BQ, MEM), jnp.float32)],
    )
    topo = pl.pallas_call(
        _gather_kernel,
        grid_spec=grid_spec,
        out_shape=jax.ShapeDtypeStruct((BQ, 1), jnp.float32),
    )(cur.reshape(BQ), cur, pr, pc, pj, ap, dones_i, adjacency_matrix)
    return topo


# speedup vs baseline: 44.3986x; 1.0461x over previous
import jax
import jax.numpy as jnp
from jax.experimental import pallas as pl
from jax.experimental.pallas import tpu as pltpu

MEM = 4096
FEAT = 512
BQ = 16
NP = 2 * BQ
TAU = 0.85
DMIN = 1.5
EPS = 1e-8
ROW_TILE = 1024


def _simnorm_kernel(nf_ref, ht_ref, st_ref, fn2_ref):
    nf = nf_ref[...]
    dn = (((1,), (1,)), ((), ()))
    st_ref[...] = jax.lax.dot_general(
        ht_ref[...], nf, dimension_numbers=dn,
        preferred_element_type=jnp.float32)
    fn2_ref[...] = jax.lax.dot_general(
        jnp.ones((1, FEAT), jnp.float32), nf * nf, dimension_numbers=dn,
        preferred_element_type=jnp.float32)


def _decide_kernel(st_ref, fn2_ref, ht_ref, htT_ref, cpos_ref, npos_ref,
                   dones_ref, lv_ref, scal_ref, cur_ref, pr_ref, pc_ref,
                   pj_ref, ap_ref):
    ST = st_ref[...]
    fn2 = fn2_ref[...]
    ht = ht_ref[...]
    G = jnp.dot(ht, htT_ref[...], preferred_element_type=jnp.float32)
    cposT = cpos_ref[...]
    nposT = npos_ref[...]
    dones = dones_ref[...]
    lv0 = lv_ref[...]
    ptr = scal_ref[0, 0]
    nn = scal_ref[0, 1]
    iota = jax.lax.broadcasted_iota(jnp.int32, (1, MEM), 1)
    iota16 = jax.lax.broadcasted_iota(jnp.int32, (BQ, 1), 0)
    iota32 = jax.lax.broadcasted_iota(jnp.int32, (NP, 1), 0)
    fn = jnp.maximum(jnp.sqrt(fn2), EPS)
    neg_inf = jnp.float32(-jnp.inf)

    ev_row = []
    ev_a = []
    ev_b = []
    ev_pos = []
    ev_srow = []
    ev_n2 = []
    pair_r, pair_c, pair_j, pair_ap = [], [], [], []
    cur_list = []

    for i in range(BQ):
        done = dones[i, 0] != 0
        hn = jnp.maximum(jnp.sqrt(G[i, i]), EPS)
        col = ST[i:i + 1, :] / (fn * hn)
        for j in range(i):
            dv = ev_a[j] * ev_srow[j][i, 0] + jnp.sum(ev_b[j] * G[:, i][None, :])
            psim = dv / (jnp.maximum(jnp.sqrt(ev_n2[j]), EPS) * hn)
            col = jnp.where(iota == ev_row[j], psim, col)
        col = jnp.where(iota < nn, col, neg_inf)
        msim = jnp.max(col)
        msi = jnp.min(jnp.where(col == msim, iota, MEM)).astype(jnp.int32)
        pos_m = jnp.sum(jnp.where(iota == msi, nposT, 0.0), axis=1,
                        keepdims=True)
        for j in range(i):
            pos_m = jnp.where(ev_row[j] == msi, ev_pos[j], pos_m)
        dpos = cposT[:, i:i + 1] - pos_m
        dist = jnp.sqrt(jnp.sum(dpos * dpos))
        should_add = (msim < TAU) | (dist > DMIN)
        empty = (~done) & (nn < 1)
        active = (~done) & (nn >= 1)
        caseA = active & should_add & (nn < MEM)
        caseB = active & should_add & (nn >= MEM)
        caseC = active & (~should_add)
        write_idx = jnp.where(
            empty, 0,
            jnp.where(caseA, nn, jnp.where(caseB, ptr, msi))).astype(jnp.int32)
        a_cur = jnp.float32(1.0)
        b_cur = jnp.zeros((1, BQ), jnp.float32)
        for j in range(i):
            m = ev_row[j] == write_idx
            a_cur = jnp.where(m, ev_a[j], a_cur)
            b_cur = jnp.where(m, ev_b[j], b_cur)
        ei = (jax.lax.broadcasted_iota(jnp.int32, (1, BQ), 1) == i
              ).astype(jnp.float32)
        a_new = jnp.where(caseC, 0.5 * a_cur, 0.0)
        b_new = jnp.where(caseC, 0.5 * b_cur, 0.0) + jnp.where(caseC, 0.5, 1.0) * ei
        wpos = jnp.where(caseC, 0.5 * pos_m + 0.5 * cposT[:, i:i + 1],
                         cposT[:, i:i + 1])
        srow_w = jnp.sum(jnp.where(iota == write_idx, ST, 0.0), axis=1,
                         keepdims=True)
        fn2_w = jnp.sum(jnp.where(iota == write_idx, fn2, 0.0))
        bG = jnp.dot(b_new, G, preferred_element_type=jnp.float32)
        n2_new = (a_new * a_new * fn2_w
                  + 2.0 * a_new * jnp.dot(b_new, srow_w,
                                          preferred_element_type=jnp.float32)[0, 0]
                  + jnp.sum(bG * b_new))
        ev_row.append(jnp.where(done, -1, write_idx))
        ev_a.append(a_new)
        ev_b.append(b_new)
        ev_pos.append(wpos)
        ev_srow.append(srow_w)
        ev_n2.append(n2_new)
        nn = nn + jnp.where(empty | caseA, 1, 0)
        ptr = jnp.where(caseB, (ptr + 1) & (MEM - 1), ptr)
        cur = write_idx
        last_idx = jnp.where(empty, 0, lv0[i, 0])
        edge = (~done) & (last_idx != -1) & (last_idx != cur)
        for (r, c) in ((last_idx, cur), (cur, last_idx)):
            dup = jnp.bool_(False)
            for m in range(len(pair_r)):
                dup = dup | (pair_ap[m] & (pair_r[m] == r) & (pair_c[m] == c))
            pair_r.append(r)
            pair_c.append(c)
            pair_j.append(jnp.int32(i))
            pair_ap.append(edge & (~dup))
        cur_list.append(cur)

    cur_arr = jnp.zeros((BQ, 1), jnp.int32)
    for i in range(BQ):
        cur_arr = jnp.where(iota16 == i, cur_list[i], cur_arr)
    cur_ref[...] = cur_arr
    pr_a = jnp.zeros((NP, 1), jnp.int32)
    pc_a = jnp.zeros((NP, 1), jnp.int32)
    pj_a = jnp.zeros((NP, 1), jnp.int32)
    ap_a = jnp.zeros((NP, 1), jnp.int32)
    for k in range(NP):
        mk = iota32 == k
        pr_a = jnp.where(mk, pair_r[k], pr_a)
        pc_a = jnp.where(mk, pair_c[k], pc_a)
        pj_a = jnp.where(mk, pair_j[k], pj_a)
        ap_a = jnp.where(mk, pair_ap[k].astype(jnp.int32), ap_a)
    pr_ref[...] = pr_a
    pc_ref[...] = pc_a
    pj_ref[...] = pj_a
    ap_ref[...] = ap_a


def _gather_kernel(cur_sref, curv_ref, pr_ref, pc_ref, pj_ref, ap_ref,
                   dn_ref, adj_ref, topo_ref, rows_ref):
    i = pl.program_id(0)
    sub = cur_sref[i] & 7
    iota8 = jax.lax.broadcasted_iota(jnp.int32, (8, 1), 0)
    rows_ref[pl.ds(i, 1), :] = jnp.sum(
        jnp.where(iota8 == sub, adj_ref[...], 0.0), axis=0, keepdims=True)

    @pl.when(i == BQ - 1)
    def _assemble():
        rows = rows_ref[...]
        sums = jnp.sum(rows, axis=1, keepdims=True)
        curT = jnp.reshape(curv_ref[...], (1, BQ))
        pr = pr_ref[...]
        pc = pc_ref[...]
        pj = pj_ref[...]
        ap = ap_ref[...]
        m = (pr == curT).astype(jnp.float32)
        count = jnp.sum(m, axis=1, keepdims=True)
        sel = jnp.dot(m, rows, preferred_element_type=jnp.float32)
        iota_c = jax.lax.broadcasted_iota(jnp.int32, (1, MEM), 1)
        onehot_c = (iota_c == pc).astype(jnp.float32)
        bval = (jnp.sum(sel * onehot_c, axis=1, keepdims=True)
                / jnp.maximum(count, 1.0))
        iota_q = jax.lax.broadcasted_iota(jnp.int32, (1, BQ), 1)
        mask = ((ap != 0) & (pr == curT) & (pj <= iota_q)).astype(jnp.float32)
        corr = jnp.sum(mask * (1.0 - bval), axis=0, keepdims=True)
        corr16 = jnp.reshape(corr, (BQ, 1))
        topo = jnp.where(dn_ref[...] != 0, 0.0, sums + corr16)
        topo_ref[...] = topo


def kernel(h_t, current_position, dones, node_features, node_positions,
           adjacency_matrix, ptr, num_nodes, last_visited_node_idx):
    htT = h_t.T
    ST, fn2 = pl.pallas_call(
        _simnorm_kernel,
        grid=(MEM // ROW_TILE,),
        in_specs=[
            pl.BlockSpec((ROW_TILE, FEAT), lambda i: (i, 0)),
            pl.BlockSpec((BQ, FEAT), lambda i: (0, 0)),
        ],
        out_specs=[
            pl.BlockSpec((BQ, ROW_TILE), lambda i: (0, i)),
            pl.BlockSpec((1, ROW_TILE), lambda i: (0, i)),
        ],
        out_shape=[
            jax.ShapeDtypeStruct((BQ, MEM), jnp.float32),
            jax.ShapeDtypeStruct((1, MEM), jnp.float32),
        ],
    )(node_features, h_t)

    dones_i = dones.astype(jnp.int32).reshape(BQ, 1)
    lv = last_visited_node_idx.astype(jnp.int32).reshape(BQ, 1)
    scal = jnp.stack([ptr.astype(jnp.int32),
                      num_nodes.astype(jnp.int32)]).reshape(1, 2)
    cur, pr, pc, pj, ap = pl.pallas_call(
        _decide_kernel,
        out_shape=[
            jax.ShapeDtypeStruct((BQ, 1), jnp.int32),
            jax.ShapeDtypeStruct((NP, 1), jnp.int32),
            jax.ShapeDtypeStruct((NP, 1), jnp.int32),
            jax.ShapeDtypeStruct((NP, 1), jnp.int32),
            jax.ShapeDtypeStruct((NP, 1), jnp.int32),
        ],
    )(ST, fn2, h_t, htT, current_position.T, node_positions.T, dones_i, lv,
      scal)

    full = lambda i, cur_s: (0, 0)
    grid_spec = pltpu.PrefetchScalarGridSpec(
        num_scalar_prefetch=1,
        grid=(BQ,),
        in_specs=[
            pl.BlockSpec((BQ, 1), full),
            pl.BlockSpec((NP, 1), full),
            pl.BlockSpec((NP, 1), full),
            pl.BlockSpec((NP, 1), full),
            pl.BlockSpec((NP, 1), full),
            pl.BlockSpec((BQ, 1), full),
            pl.BlockSpec((8, MEM), lambda i, cur_s: (cur_s[i] // 8, 0)),
        ],
        out_specs=pl.BlockSpec((BQ, 1), full),
        scratch_shapes=[pltpu.VMEM((BQ, MEM), jnp.float32)],
    )
    topo = pl.pallas_call(
        _gather_kernel,
        grid_spec=grid_spec,
        out_shape=jax.ShapeDtypeStruct((BQ, 1), jnp.float32),
    )(cur.reshape(BQ), cur, pr, pc, pj, ap, dones_i, adjacency_matrix)
    return topo


# speedup vs baseline: 45.1140x; 1.0161x over previous
import jax
import jax.numpy as jnp
from jax.experimental import pallas as pl
from jax.experimental.pallas import tpu as pltpu

MEM = 4096
FEAT = 512
BQ = 16
NP = 2 * BQ
TAU = 0.85
DMIN = 1.5
EPS = 1e-8
ROW_TILE = 2048


def _simnorm_kernel(nf_ref, ht_ref, st_ref, fn2_ref):
    nf = nf_ref[...]
    dn = (((1,), (1,)), ((), ()))
    st_ref[...] = jax.lax.dot_general(
        ht_ref[...], nf, dimension_numbers=dn,
        preferred_element_type=jnp.float32)
    fn2_ref[...] = jax.lax.dot_general(
        jnp.ones((1, FEAT), jnp.float32), nf * nf, dimension_numbers=dn,
        preferred_element_type=jnp.float32)


def _decide_kernel(st_ref, fn2_ref, ht_ref, htT_ref, cpos_ref, npos_ref,
                   dones_ref, lv_ref, scal_ref, cur_ref, pr_ref, pc_ref,
                   pj_ref, ap_ref):
    ST = st_ref[...]
    fn2 = fn2_ref[...]
    ht = ht_ref[...]
    G = jnp.dot(ht, htT_ref[...], preferred_element_type=jnp.float32)
    cposT = cpos_ref[...]
    nposT = npos_ref[...]
    dones = dones_ref[...]
    lv0 = lv_ref[...]
    ptr = scal_ref[0, 0]
    nn = scal_ref[0, 1]
    iota = jax.lax.broadcasted_iota(jnp.int32, (1, MEM), 1)
    iota16 = jax.lax.broadcasted_iota(jnp.int32, (BQ, 1), 0)
    iota32 = jax.lax.broadcasted_iota(jnp.int32, (NP, 1), 0)
    fn = jnp.maximum(jnp.sqrt(fn2), EPS)
    neg_inf = jnp.float32(-jnp.inf)

    ev_row = []
    ev_a = []
    ev_b = []
    ev_pos = []
    ev_srow = []
    ev_n2 = []
    pair_r, pair_c, pair_j, pair_ap = [], [], [], []
    cur_list = []

    for i in range(BQ):
        done = dones[i, 0] != 0
        hn = jnp.maximum(jnp.sqrt(G[i, i]), EPS)
        col = ST[i:i + 1, :] / (fn * hn)
        for j in range(i):
            dv = ev_a[j] * ev_srow[j][i, 0] + jnp.sum(ev_b[j] * G[:, i][None, :])
            psim = dv / (jnp.maximum(jnp.sqrt(ev_n2[j]), EPS) * hn)
            col = jnp.where(iota == ev_row[j], psim, col)
        col = jnp.where(iota < nn, col, neg_inf)
        msim = jnp.max(col)
        msi = jnp.min(jnp.where(col == msim, iota, MEM)).astype(jnp.int32)
        pos_m = jnp.sum(jnp.where(iota == msi, nposT, 0.0), axis=1,
                        keepdims=True)
        for j in range(i):
            pos_m = jnp.where(ev_row[j] == msi, ev_pos[j], pos_m)
        dpos = cposT[:, i:i + 1] - pos_m
        dist = jnp.sqrt(jnp.sum(dpos * dpos))
        should_add = (msim < TAU) | (dist > DMIN)
        empty = (~done) & (nn < 1)
        active = (~done) & (nn >= 1)
        caseA = active & should_add & (nn < MEM)
        caseB = active & should_add & (nn >= MEM)
        caseC = active & (~should_add)
        write_idx = jnp.where(
            empty, 0,
            jnp.where(caseA, nn, jnp.where(caseB, ptr, msi))).astype(jnp.int32)
        a_cur = jnp.float32(1.0)
        b_cur = jnp.zeros((1, BQ), jnp.float32)
        for j in range(i):
            m = ev_row[j] == write_idx
            a_cur = jnp.where(m, ev_a[j], a_cur)
            b_cur = jnp.where(m, ev_b[j], b_cur)
        ei = (jax.lax.broadcasted_iota(jnp.int32, (1, BQ), 1) == i
              ).astype(jnp.float32)
        a_new = jnp.where(caseC, 0.5 * a_cur, 0.0)
        b_new = jnp.where(caseC, 0.5 * b_cur, 0.0) + jnp.where(caseC, 0.5, 1.0) * ei
        wpos = jnp.where(caseC, 0.5 * pos_m + 0.5 * cposT[:, i:i + 1],
                         cposT[:, i:i + 1])
        srow_w = jnp.sum(jnp.where(iota == write_idx, ST, 0.0), axis=1,
                         keepdims=True)
        fn2_w = jnp.sum(jnp.where(iota == write_idx, fn2, 0.0))
        bG = jnp.dot(b_new, G, preferred_element_type=jnp.float32)
        n2_new = (a_new * a_new * fn2_w
                  + 2.0 * a_new * jnp.dot(b_new, srow_w,
                                          preferred_element_type=jnp.float32)[0, 0]
                  + jnp.sum(bG * b_new))
        ev_row.append(jnp.where(done, -1, write_idx))
        ev_a.append(a_new)
        ev_b.append(b_new)
        ev_pos.append(wpos)
        ev_srow.append(srow_w)
        ev_n2.append(n2_new)
        nn = nn + jnp.where(empty | caseA, 1, 0)
        ptr = jnp.where(caseB, (ptr + 1) & (MEM - 1), ptr)
        cur = write_idx
        last_idx = jnp.where(empty, 0, lv0[i, 0])
        edge = (~done) & (last_idx != -1) & (last_idx != cur)
        for (r, c) in ((last_idx, cur), (cur, last_idx)):
            dup = jnp.bool_(False)
            for m in range(len(pair_r)):
                dup = dup | (pair_ap[m] & (pair_r[m] == r) & (pair_c[m] == c))
            pair_r.append(r)
            pair_c.append(c)
            pair_j.append(jnp.int32(i))
            pair_ap.append(edge & (~dup))
        cur_list.append(cur)

    cur_arr = jnp.zeros((BQ, 1), jnp.int32)
    for i in range(BQ):
        cur_arr = jnp.where(iota16 == i, cur_list[i], cur_arr)
    cur_ref[...] = cur_arr
    pr_a = jnp.zeros((NP, 1), jnp.int32)
    pc_a = jnp.zeros((NP, 1), jnp.int32)
    pj_a = jnp.zeros((NP, 1), jnp.int32)
    ap_a = jnp.zeros((NP, 1), jnp.int32)
    for k in range(NP):
        mk = iota32 == k
        pr_a = jnp.where(mk, pair_r[k], pr_a)
        pc_a = jnp.where(mk, pair_c[k], pc_a)
        pj_a = jnp.where(mk, pair_j[k], pj_a)
        ap_a = jnp.where(mk, pair_ap[k].astype(jnp.int32), ap_a)
    pr_ref[...] = pr_a
    pc_ref[...] = pc_a
    pj_ref[...] = pj_a
    ap_ref[...] = ap_a


def _gather_kernel(cur_sref, curv_ref, pr_ref, pc_ref, pj_ref, ap_ref,
                   dn_ref, adj_ref, topo_ref, rows_ref):
    i = pl.program_id(0)
    sub = cur_sref[i] & 7
    iota8 = jax.lax.broadcasted_iota(jnp.int32, (8, 1), 0)
    rows_ref[pl.ds(i, 1), :] = jnp.sum(
        jnp.where(iota8 == sub, adj_ref[...], 0.0), axis=0, keepdims=True)

    @pl.when(i == BQ - 1)
    def _assemble():
        rows = rows_ref[...]
        sums = jnp.sum(rows, axis=1, keepdims=True)
        curT = jnp.reshape(curv_ref[...], (1, BQ))
        pr = pr_ref[...]
        pc = pc_ref[...]
        pj = pj_ref[...]
        ap = ap_ref[...]
        m = (pr == curT).astype(jnp.float32)
        count = jnp.sum(m, axis=1, keepdims=True)
        sel = jnp.dot(m, rows, preferred_element_type=jnp.float32)
        iota_c = jax.lax.broadcasted_iota(jnp.int32, (1, MEM), 1)
        onehot_c = (iota_c == pc).astype(jnp.float32)
        bval = (jnp.sum(sel * onehot_c, axis=1, keepdims=True)
                / jnp.maximum(count, 1.0))
        iota_q = jax.lax.broadcasted_iota(jnp.int32, (1, BQ), 1)
        mask = ((ap != 0) & (pr == curT) & (pj <= iota_q)).astype(jnp.float32)
        corr = jnp.sum(mask * (1.0 - bval), axis=0, keepdims=True)
        corr16 = jnp.reshape(corr, (BQ, 1))
        topo = jnp.where(dn_ref[...] != 0, 0.0, sums + corr16)
        topo_ref[...] = topo


def kernel(h_t, current_position, dones, node_features, node_positions,
           adjacency_matrix, ptr, num_nodes, last_visited_node_idx):
    htT = h_t.T
    ST, fn2 = pl.pallas_call(
        _simnorm_kernel,
        grid=(MEM // ROW_TILE,),
        in_specs=[
            pl.BlockSpec((ROW_TILE, FEAT), lambda i: (i, 0)),
            pl.BlockSpec((BQ, FEAT), lambda i: (0, 0)),
        ],
        out_specs=[
            pl.BlockSpec((BQ, ROW_TILE), lambda i: (0, i)),
            pl.BlockSpec((1, ROW_TILE), lambda i: (0, i)),
        ],
        out_shape=[
            jax.ShapeDtypeStruct((BQ, MEM), jnp.float32),
            jax.ShapeDtypeStruct((1, MEM), jnp.float32),
        ],
    )(node_features, h_t)

    dones_i = dones.astype(jnp.int32).reshape(BQ, 1)
    lv = last_visited_node_idx.astype(jnp.int32).reshape(BQ, 1)
    scal = jnp.stack([ptr.astype(jnp.int32),
                      num_nodes.astype(jnp.int32)]).reshape(1, 2)
    cur, pr, pc, pj, ap = pl.pallas_call(
        _decide_kernel,
        out_shape=[
            jax.ShapeDtypeStruct((BQ, 1), jnp.int32),
            jax.ShapeDtypeStruct((NP, 1), jnp.int32),
            jax.ShapeDtypeStruct((NP, 1), jnp.int32),
            jax.ShapeDtypeStruct((NP, 1), jnp.int32),
            jax.ShapeDtypeStruct((NP, 1), jnp.int32),
        ],
    )(ST, fn2, h_t, htT, current_position.T, node_positions.T, dones_i, lv,
      scal)

    full = lambda i, cur_s: (0, 0)
    grid_spec = pltpu.PrefetchScalarGridSpec(
        num_scalar_prefetch=1,
        grid=(BQ,),
        in_specs=[
            pl.BlockSpec((BQ, 1), full),
            pl.BlockSpec((NP, 1), full),
            pl.BlockSpec((NP, 1), full),
            pl.BlockSpec((NP, 1), full),
            pl.BlockSpec((NP, 1), full),
            pl.BlockSpec((BQ, 1), full),
            pl.BlockSpec((8, MEM), lambda i, cur_s: (cur_s[i] // 8, 0)),
        ],
        out_specs=pl.BlockSpec((BQ, 1), full),
        scratch_shapes=[pltpu.VMEM((BQ, MEM), jnp.float32)],
    )
    topo = pl.pallas_call(
        _gather_kernel,
        grid_spec=grid_spec,
        out_shape=jax.ShapeDtypeStruct((BQ, 1), jnp.float32),
    )(cur.reshape(BQ), cur, pr, pc, pj, ap, dones_i, adjacency_matrix)
    return topo


# speedup vs baseline: 50.0915x; 1.1103x over previous
import jax
import jax.numpy as jnp
from jax.experimental import pallas as pl
from jax.experimental.pallas import tpu as pltpu

MEM = 4096
FEAT = 512
BQ = 16
NP = 2 * BQ
TAU = 0.85
DMIN = 1.5
EPS = 1e-8
ROW_TILE = 2048
NTILES = MEM // ROW_TILE
_CONTRACT1 = (((1,), (1,)), ((), ()))


def _decide_body(ST, fn2, G, cposT, nposT, dones, lv0, scal_ref,
                 cur_ref, pr_ref, pc_ref, pj_ref, ap_ref):
    ptr = scal_ref[0, 0]
    nn = scal_ref[0, 1]
    iota = jax.lax.broadcasted_iota(jnp.int32, (1, MEM), 1)
    iota16 = jax.lax.broadcasted_iota(jnp.int32, (BQ, 1), 0)
    iota16l = jax.lax.broadcasted_iota(jnp.int32, (BQ, BQ), 1)
    iota32 = jax.lax.broadcasted_iota(jnp.int32, (NP, 1), 0)
    fn = jnp.maximum(jnp.sqrt(fn2), EPS)
    diag = jnp.sum(jnp.where(iota16 == iota16l, G, 0.0), axis=1,
                   keepdims=True)
    hnv = jnp.maximum(jnp.sqrt(diag), EPS)
    SIMS = ST / (fn * hnv)
    neg_inf = jnp.float32(-jnp.inf)

    ev_row = []
    ev_a = []
    ev_b = []
    ev_pos = []
    ev_srow = []
    ev_n2 = []
    pair_r, pair_c, pair_j, pair_ap = [], [], [], []
    cur_list = []

    for i in range(BQ):
        done = dones[i, 0] != 0
        hn = hnv[i, 0]
        col = SIMS[i:i + 1, :]
        for j in range(i):
            dv = ev_a[j] * ev_srow[j][i, 0] + jnp.sum(ev_b[j] * G[:, i][None, :])
            psim = dv / (jnp.maximum(jnp.sqrt(ev_n2[j]), EPS) * hn)
            col = jnp.where(iota == ev_row[j], psim, col)
        col = jnp.where(iota < nn, col, neg_inf)
        msim = jnp.max(col)
        msi = jnp.min(jnp.where(col == msim, iota, MEM)).astype(jnp.int32)
        pos_m = jnp.sum(jnp.where(iota == msi, nposT, 0.0), axis=1,
                        keepdims=True)
        for j in range(i):
            pos_m = jnp.where(ev_row[j] == msi, ev_pos[j], pos_m)
        dpos = cposT[:, i:i + 1] - pos_m
        dist = jnp.sqrt(jnp.sum(dpos * dpos))
        should_add = (msim < TAU) | (dist > DMIN)
        empty = (~done) & (nn < 1)
        active = (~done) & (nn >= 1)
        caseA = active & should_add & (nn < MEM)
        caseB = active & should_add & (nn >= MEM)
        caseC = active & (~should_add)
        write_idx = jnp.where(
            empty, 0,
            jnp.where(caseA, nn, jnp.where(caseB, ptr, msi))).astype(jnp.int32)
        a_cur = jnp.float32(1.0)
        b_cur = jnp.zeros((1, BQ), jnp.float32)
        for j in range(i):
            m = ev_row[j] == write_idx
            a_cur = jnp.where(m, ev_a[j], a_cur)
            b_cur = jnp.where(m, ev_b[j], b_cur)
        ei = (jax.lax.broadcasted_iota(jnp.int32, (1, BQ), 1) == i
              ).astype(jnp.float32)
        a_new = jnp.where(caseC, 0.5 * a_cur, 0.0)
        b_new = jnp.where(caseC, 0.5 * b_cur, 0.0) + jnp.where(caseC, 0.5, 1.0) * ei
        wpos = jnp.where(caseC, 0.5 * pos_m + 0.5 * cposT[:, i:i + 1],
                         cposT[:, i:i + 1])
        srow_w = jnp.sum(jnp.where(iota == write_idx, ST, 0.0), axis=1,
                         keepdims=True)
        fn2_w = jnp.sum(jnp.where(iota == write_idx, fn2, 0.0))
        bG = jnp.dot(b_new, G, preferred_element_type=jnp.float32)
        n2_new = (a_new * a_new * fn2_w
                  + 2.0 * a_new * jnp.dot(b_new, srow_w,
                                          preferred_element_type=jnp.float32)[0, 0]
                  + jnp.sum(bG * b_new))
        ev_row.append(jnp.where(done, -1, write_idx))
        ev_a.append(a_new)
        ev_b.append(b_new)
        ev_pos.append(wpos)
        ev_srow.append(srow_w)
        ev_n2.append(n2_new)
        nn = nn + jnp.where(empty | caseA, 1, 0)
        ptr = jnp.where(caseB, (ptr + 1) & (MEM - 1), ptr)
        cur = write_idx
        last_idx = jnp.where(empty, 0, lv0[i, 0])
        edge = (~done) & (last_idx != -1) & (last_idx != cur)
        for (r, c) in ((last_idx, cur), (cur, last_idx)):
            dup = jnp.bool_(False)
            for m in range(len(pair_r)):
                dup = dup | (pair_ap[m] & (pair_r[m] == r) & (pair_c[m] == c))
            pair_r.append(r)
            pair_c.append(c)
            pair_j.append(jnp.int32(i))
            pair_ap.append(edge & (~dup))
        cur_list.append(cur)

    cur_arr = jnp.zeros((BQ, 1), jnp.int32)
    for i in range(BQ):
        cur_arr = jnp.where(iota16 == i, cur_list[i], cur_arr)
    cur_ref[...] = cur_arr
    pr_a = jnp.zeros((NP, 1), jnp.int32)
    pc_a = jnp.zeros((NP, 1), jnp.int32)
    pj_a = jnp.zeros((NP, 1), jnp.int32)
    ap_a = jnp.zeros((NP, 1), jnp.int32)
    for k in range(NP):
        mk = iota32 == k
        pr_a = jnp.where(mk, pair_r[k], pr_a)
        pc_a = jnp.where(mk, pair_c[k], pc_a)
        pj_a = jnp.where(mk, pair_j[k], pj_a)
        ap_a = jnp.where(mk, pair_ap[k].astype(jnp.int32), ap_a)
    pr_ref[...] = pr_a
    pc_ref[...] = pc_a
    pj_ref[...] = pj_a
    ap_ref[...] = ap_a


def _fused_kernel(nf_ref, ht_ref, cpos_ref, npos_ref, dones_ref, lv_ref,
                  scal_ref, cur_ref, pr_ref, pc_ref, pj_ref, ap_ref,
                  st_ref, fn2_ref):
    t = pl.program_id(0)
    nf = nf_ref[...]
    ht = ht_ref[...]
    st_ref[:, pl.ds(t * ROW_TILE, ROW_TILE)] = jax.lax.dot_general(
        ht, nf, dimension_numbers=_CONTRACT1,
        preferred_element_type=jnp.float32)
    fn2_ref[:, pl.ds(t * ROW_TILE, ROW_TILE)] = jax.lax.dot_general(
        jnp.ones((1, FEAT), jnp.float32), nf * nf,
        dimension_numbers=_CONTRACT1, preferred_element_type=jnp.float32)

    @pl.when(t == NTILES - 1)
    def _decide():
        G = jax.lax.dot_general(ht, ht, dimension_numbers=_CONTRACT1,
                                preferred_element_type=jnp.float32)
        _decide_body(st_ref[...], fn2_ref[...], G, cpos_ref[...],
                     npos_ref[...], dones_ref[...], lv_ref[...], scal_ref,
                     cur_ref, pr_ref, pc_ref, pj_ref, ap_ref)


def _gather_kernel(cur_sref, curv_ref, pr_ref, pc_ref, pj_ref, ap_ref,
                   dn_ref, adj_ref, topo_ref, rows_ref):
    i = pl.program_id(0)
    sub = cur_sref[i] & 7
    iota8 = jax.lax.broadcasted_iota(jnp.int32, (8, 1), 0)
    rows_ref[pl.ds(i, 1), :] = jnp.sum(
        jnp.where(iota8 == sub, adj_ref[...], 0.0), axis=0, keepdims=True)

    @pl.when(i == BQ - 1)
    def _assemble():
        rows = rows_ref[...]
        sums = jnp.sum(rows, axis=1, keepdims=True)
        curT = jnp.reshape(curv_ref[...], (1, BQ))
        pr = pr_ref[...]
        pc = pc_ref[...]
        pj = pj_ref[...]
        ap = ap_ref[...]
        m = (pr == curT).astype(jnp.float32)
        count = jnp.sum(m, axis=1, keepdims=True)
        sel = jnp.dot(m, rows, preferred_element_type=jnp.float32)
        iota_c = jax.lax.broadcasted_iota(jnp.int32, (1, MEM), 1)
        onehot_c = (iota_c == pc).astype(jnp.float32)
        bval = (jnp.sum(sel * onehot_c, axis=1, keepdims=True)
                / jnp.maximum(count, 1.0))
        iota_q = jax.lax.broadcasted_iota(jnp.int32, (1, BQ), 1)
        mask = ((ap != 0) & (pr == curT) & (pj <= iota_q)).astype(jnp.float32)
        corr = jnp.sum(mask * (1.0 - bval), axis=0, keepdims=True)
        corr16 = jnp.reshape(corr, (BQ, 1))
        topo = jnp.where(dn_ref[...] != 0, 0.0, sums + corr16)
        topo_ref[...] = topo


def kernel(h_t, current_position, dones, node_features, node_positions,
           adjacency_matrix, ptr, num_nodes, last_visited_node_idx):
    dones_i = dones.astype(jnp.int32).reshape(BQ, 1)
    lv = last_visited_node_idx.astype(jnp.int32).reshape(BQ, 1)
    scal = jnp.stack([ptr.astype(jnp.int32),
                      num_nodes.astype(jnp.int32)]).reshape(1, 2)
    full = lambda i: (0, 0)
    cur, pr, pc, pj, ap = pl.pallas_call(
        _fused_kernel,
        grid=(NTILES,),
        in_specs=[
            pl.BlockSpec((ROW_TILE, FEAT), lambda i: (i, 0)),
            pl.BlockSpec((BQ, FEAT), full),
            pl.BlockSpec((3, BQ), full),
            pl.BlockSpec((3, MEM), full),
            pl.BlockSpec((BQ, 1), full),
            pl.BlockSpec((BQ, 1), full),
            pl.BlockSpec((1, 2), full),
        ],
        out_specs=[
            pl.BlockSpec((BQ, 1), full),
            pl.BlockSpec((NP, 1), full),
            pl.BlockSpec((NP, 1), full),
            pl.BlockSpec((NP, 1), full),
            pl.BlockSpec((NP, 1), full),
        ],
        out_shape=[
            jax.ShapeDtypeStruct((BQ, 1), jnp.int32),
            jax.ShapeDtypeStruct((NP, 1), jnp.int32),
            jax.ShapeDtypeStruct((NP, 1), jnp.int32),
            jax.ShapeDtypeStruct((NP, 1), jnp.int32),
            jax.ShapeDtypeStruct((NP, 1), jnp.int32),
        ],
        scratch_shapes=[
            pltpu.VMEM((BQ, MEM), jnp.float32),
            pltpu.VMEM((1, MEM), jnp.float32),
        ],
    )(node_features, h_t, current_position.T, node_positions.T, dones_i, lv,
      scal)

    fullp = lambda i, cur_s: (0, 0)
    grid_spec = pltpu.PrefetchScalarGridSpec(
        num_scalar_prefetch=1,
        grid=(BQ,),
        in_specs=[
            pl.BlockSpec((BQ, 1), fullp),
            pl.BlockSpec((NP, 1), fullp),
            pl.BlockSpec((NP, 1), fullp),
            pl.BlockSpec((NP, 1), fullp),
            pl.BlockSpec((NP, 1), fullp),
            pl.BlockSpec((BQ, 1), fullp),
            pl.BlockSpec((8, MEM), lambda i, cur_s: (cur_s[i] // 8, 0)),
        ],
        out_specs=pl.BlockSpec((BQ, 1), fullp),
        scratch_shapes=[pltpu.VMEM((BQ, MEM), jnp.float32)],
    )
    topo = pl.pallas_call(
        _gather_kernel,
        grid_spec=grid_spec,
        out_shape=jax.ShapeDtypeStruct((BQ, 1), jnp.float32),
    )(cur.reshape(BQ), cur, pr, pc, pj, ap, dones_i, adjacency_matrix)
    return topo
